# baseline (speedup 1.0000x reference)
import jax
import jax.numpy as jnp
from jax import lax
from jax.experimental import pallas as pl
from jax.experimental.pallas import tpu as pltpu
from jax.experimental.pallas import tpu_sc as plsc

N_NODES = 10000
GENE_NUM = 2000
N_EDGES = 320000
DIM_IN = 128
DIM_HID = 256
DIM_OUT = 32

NC = 2
NS = 16
L = 16

NPAD = 10240
STRIPE = NPAD // NS
K = 128
ET = N_EDGES // (NC * NS)
NB = 79
ETP = NB * K
E_PACK = NC * NS * ETP
ALPHA_PAD = 2048


def _scale_rows(rows_ref, c_ref, n_rows, width):
  nch = width // L

  def chunk(t, _):
    ch = c_ref[pl.ds(t * L, L)]
    for rr in range(L):
      cs = jnp.full((L,), ch[rr], dtype=jnp.float32)
      r = t * L + rr
      for j in range(nch):
        sl = pl.ds(j * L, L)
        rows_ref[r, sl] = rows_ref[r, sl] * cs
    return 0

  lax.fori_loop(0, n_rows // L, chunk, 0, unroll=4)


def _edge_coeffs(ebuf, nid_v, al_v, cv, dstb):
  for t in range(K // L):
    sl = pl.ds(t * L, L)
    sv = ebuf[0, sl]
    dv = ebuf[1, sl]
    w = plsc.bitcast(ebuf[2, sl], jnp.float32)
    sid = plsc.load_gather(nid_v, [sv])
    did = plsc.load_gather(nid_v, [dv])
    aidx = jnp.full((L,), GENE_NUM + 1, jnp.int32)
    aidx = jnp.where((sid >= 0) & (did < 0), sid, aidx)
    aidx = jnp.where((did >= 0) & (sid < 0), did, aidx)
    a = plsc.load_gather(al_v, [aidx])
    cv[sl] = w * a
    dstb[sl] = dv


def _pipeline(ep_hbm, feat_hbm, acc_sh, cnt_sh, nid_v, al_v, ebufs, dstbs,
              cv, ones_v, rows, sems_e, sems_g, sems_w, sems_c, width, base0):
  def ep_slice(b):
    return ep_hbm.at[:, pl.ds(base0 + b * K, K)]

  def fire_e(b, slot):
    pltpu.async_copy(ep_slice(b), ebufs[slot], sems_e[slot])

  def wait_e(slot):
    pltpu.make_async_copy(ep_slice(0), ebufs[slot], sems_e[slot]).wait()

  def fire_g(slot):
    pltpu.async_copy(feat_hbm.at[ebufs[slot].at[0]], rows[slot],
                     sems_g[slot])

  def wait_g(slot):
    pltpu.make_async_copy(feat_hbm.at[ebufs[slot].at[0]], rows[slot],
                          sems_g[slot]).wait()

  def fire_w(slot):
    pltpu.async_copy(rows[slot], acc_sh.at[dstbs[slot]], sems_w[slot],
                     add=True)

  def wait_w(slot):
    pltpu.make_async_copy(rows[slot], acc_sh.at[dstbs[slot]],
                          sems_w[slot]).wait()

  def fire_c(slot):
    pltpu.async_copy(ones_v, cnt_sh.at[dstbs[slot]], sems_c[slot], add=True)

  def wait_c(slot):
    pltpu.make_async_copy(ones_v, cnt_sh.at[dstbs[slot]],
                          sems_c[slot]).wait()

  def compute(slot):
    _edge_coeffs(ebufs[slot], nid_v, al_v, cv, dstbs[slot])
    if cnt_sh is not None:
      fire_c(slot)
    wait_g(slot)
    _scale_rows(rows[slot], cv, K, width)
    fire_w(slot)

  fire_e(0, 0)
  wait_e(0)
  fire_g(0)
  fire_e(1, 1)
  compute(0)
  fire_e(2, 0)
  wait_e(1)
  fire_g(1)
  compute(1)
  fire_e(3, 1)
  wait_e(0)
  wait_w(0)
  fire_g(0)

  def sub(b, slot):
    if cnt_sh is not None:
      wait_c(slot)
    compute(slot)
    fire_e(b + 2, slot)
    wait_e(1 - slot)
    wait_w(1 - slot)
    fire_g(1 - slot)

  def body(i, _):
    sub(2 * i + 2, 0)
    sub(2 * i + 3, 1)
    return 0

  lax.fori_loop(0, (NB - 3) // 2, body, 0)

  if cnt_sh is not None:
    wait_c(0)
  compute(0)
  wait_e(1)
  wait_w(0)
  wait_w(1)
  if cnt_sh is not None:
    wait_c(0)
    wait_c(1)


def _agg1_body(ep_hbm, nid_hbm, al_hbm, x_hbm, z128_hbm, z1_hbm,
               p0_hbm, p1_hbm, cp0_hbm, cp1_hbm,
               t1_sh, cnt_sh, ebuf0, ebuf1, dstb0, dstb1, cv, ones_v,
               rows0, rows1, nid_v, al_v,
               se0, se1, sg0, sg1, sw0, sw1, sc0, sc1):
  core = lax.axis_index("c")
  s = lax.axis_index("s")

  pltpu.sync_copy(z128_hbm, t1_sh.at[pl.ds(s * STRIPE, STRIPE)])
  pltpu.sync_copy(z1_hbm, cnt_sh.at[pl.ds(s * STRIPE, STRIPE)])
  for i in range(K // L):
    ones_v[pl.ds(i * L, L)] = jnp.ones((L,), jnp.float32)
  pltpu.sync_copy(nid_hbm, nid_v)
  pltpu.sync_copy(al_hbm, al_v)
  plsc.subcore_barrier()

  base0 = (core * NS + s) * ETP
  _pipeline(ep_hbm, x_hbm, t1_sh, cnt_sh, nid_v, al_v, (ebuf0, ebuf1),
            (dstb0, dstb1), cv, ones_v, (rows0, rows1), (se0, se1),
            (sg0, sg1), (sw0, sw1), (sc0, sc1), DIM_IN, base0)
  plsc.subcore_barrier()

  stripe = pl.ds(s * STRIPE, STRIPE)

  @pl.when(core == 0)
  def _():
    pltpu.sync_copy(t1_sh.at[stripe], p0_hbm.at[stripe])
    pltpu.sync_copy(cnt_sh.at[stripe], cp0_hbm.at[stripe])

  @pl.when(core == 1)
  def _():
    pltpu.sync_copy(t1_sh.at[stripe], p1_hbm.at[stripe])
    pltpu.sync_copy(cnt_sh.at[stripe], cp1_hbm.at[stripe])


def _agg2_body(ep_hbm, nid_hbm, al_hbm, u_hbm, z32_hbm,
               op0_hbm, op1_hbm,
               out_sh, ebuf0, ebuf1, dstb0, dstb1, cv, ones_v, rows0, rows1,
               nid_v, al_v, se0, se1, sg0, sg1, sw0, sw1):
  core = lax.axis_index("c")
  s = lax.axis_index("s")

  pltpu.sync_copy(z32_hbm, out_sh.at[pl.ds(s * STRIPE, STRIPE)])
  pltpu.sync_copy(nid_hbm, nid_v)
  pltpu.sync_copy(al_hbm, al_v)
  plsc.subcore_barrier()

  base0 = (core * NS + s) * ETP
  _pipeline(ep_hbm, u_hbm, out_sh, None, nid_v, al_v, (ebuf0, ebuf1),
            (dstb0, dstb1), cv, ones_v, (rows0, rows1), (se0, se1),
            (sg0, sg1), (sw0, sw1), None, DIM_OUT, base0)
  plsc.subcore_barrier()

  stripe = pl.ds(s * STRIPE, STRIPE)

  @pl.when(core == 0)
  def _():
    pltpu.sync_copy(out_sh.at[stripe], op0_hbm.at[stripe])

  @pl.when(core == 1)
  def _():
    pltpu.sync_copy(out_sh.at[stripe], op1_hbm.at[stripe])


def _sc_agg1(ep, node_id, alpha_pad, x, z128, z1):
  mesh = plsc.VectorSubcoreMesh(core_axis_name="c", subcore_axis_name="s")
  f = pl.kernel(
      _agg1_body,
      out_type=(
          jax.ShapeDtypeStruct((NPAD, DIM_IN), jnp.float32),
          jax.ShapeDtypeStruct((NPAD, DIM_IN), jnp.float32),
          jax.ShapeDtypeStruct((NPAD,), jnp.float32),
          jax.ShapeDtypeStruct((NPAD,), jnp.float32),
      ),
      mesh=mesh,
      scratch_types=[
          pltpu.VMEM_SHARED((NPAD, DIM_IN), jnp.float32),
          pltpu.VMEM_SHARED((NPAD,), jnp.float32),
          pltpu.VMEM((3, K), jnp.int32),
          pltpu.VMEM((3, K), jnp.int32),
          pltpu.VMEM((K,), jnp.int32),
          pltpu.VMEM((K,), jnp.int32),
          pltpu.VMEM((K,), jnp.float32),
          pltpu.VMEM((K,), jnp.float32),
          pltpu.VMEM((K, DIM_IN), jnp.float32),
          pltpu.VMEM((K, DIM_IN), jnp.float32),
          pltpu.VMEM((N_NODES,), jnp.int32),
          pltpu.VMEM((ALPHA_PAD,), jnp.float32),
      ] + [pltpu.SemaphoreType.DMA] * 8,
      compiler_params=pltpu.CompilerParams(needs_layout_passes=False,
                                           use_tc_tiling_on_sc=False),
  )
  return f(ep, node_id, alpha_pad, x, z128, z1)


def _sc_agg2(ep, node_id, alpha_pad, u, z32):
  mesh = plsc.VectorSubcoreMesh(core_axis_name="c", subcore_axis_name="s")
  f = pl.kernel(
      _agg2_body,
      out_type=(
          jax.ShapeDtypeStruct((NPAD, DIM_OUT), jnp.float32),
          jax.ShapeDtypeStruct((NPAD, DIM_OUT), jnp.float32),
      ),
      mesh=mesh,
      scratch_types=[
          pltpu.VMEM_SHARED((NPAD, DIM_OUT), jnp.float32),
          pltpu.VMEM((3, K), jnp.int32),
          pltpu.VMEM((3, K), jnp.int32),
          pltpu.VMEM((K,), jnp.int32),
          pltpu.VMEM((K,), jnp.int32),
          pltpu.VMEM((K,), jnp.float32),
          pltpu.VMEM((K,), jnp.float32),
          pltpu.VMEM((K, DIM_OUT), jnp.float32),
          pltpu.VMEM((K, DIM_OUT), jnp.float32),
          pltpu.VMEM((N_NODES,), jnp.int32),
          pltpu.VMEM((ALPHA_PAD,), jnp.float32),
      ] + [pltpu.SemaphoreType.DMA] * 6,
      compiler_params=pltpu.CompilerParams(needs_layout_passes=False,
                                           use_tc_tiling_on_sc=False),
  )
  return f(ep, node_id, alpha_pad, u, z32)


def _tc_dense_kern(p0_ref, p1_ref, c0_ref, c1_ref, w1_ref, b1_ref, w2_ref,
                   wo_ref, u_ref):
  r = 1.0 / jnp.maximum(c0_ref[...] + c1_ref[...], 1.0)
  t1 = (p0_ref[...] + p1_ref[...]) * r
  h2 = lax.dot_general(t1, w1_ref[...], (((1,), (1,)), ((), ())),
                       preferred_element_type=jnp.float32) + b1_ref[...]
  wc = lax.dot_general(wo_ref[...], w2_ref[...], (((1,), (0,)), ((), ())),
                       preferred_element_type=jnp.float32)
  u_ref[...] = lax.dot_general(h2, wc, (((1,), (1,)), ((), ())),
                               preferred_element_type=jnp.float32)


def _tc_dense(p0, p1, c0, c1, W1, b1r, W2, Wout):
  bm = 512
  grid = (NPAD // bm,)
  return pl.pallas_call(
      _tc_dense_kern,
      grid=grid,
      in_specs=[
          pl.BlockSpec((bm, DIM_IN), lambda i: (i, 0)),
          pl.BlockSpec((bm, DIM_IN), lambda i: (i, 0)),
          pl.BlockSpec((bm, 1), lambda i: (i, 0)),
          pl.BlockSpec((bm, 1), lambda i: (i, 0)),
          pl.BlockSpec((DIM_HID, DIM_IN), lambda i: (0, 0)),
          pl.BlockSpec((1, DIM_HID), lambda i: (0, 0)),
          pl.BlockSpec((DIM_HID, DIM_HID), lambda i: (0, 0)),
          pl.BlockSpec((DIM_OUT, DIM_HID), lambda i: (0, 0)),
      ],
      out_specs=pl.BlockSpec((bm, DIM_OUT), lambda i: (i, 0)),
      out_shape=jax.ShapeDtypeStruct((NPAD, DIM_OUT), jnp.float32),
  )(p0, p1, c0, c1, W1, b1r, W2, Wout)


def _tc_final_kern(p0_ref, p1_ref, c0_ref, c1_ref, wo_ref, b2_ref, bo_ref,
                   out_ref):
  r = 1.0 / jnp.maximum(c0_ref[...] + c1_ref[...], 1.0)
  bc = lax.dot_general(b2_ref[...], wo_ref[...], (((1,), (1,)), ((), ())),
                       preferred_element_type=jnp.float32)
  out_ref[...] = (p0_ref[...] + p1_ref[...]) * r + bc + bo_ref[...]


def _tc_final(op0, op1, c0, c1, Wout, b2r, boutr):
  bm = 400
  grid = (N_NODES // bm,)
  return pl.pallas_call(
      _tc_final_kern,
      grid=grid,
      in_specs=[
          pl.BlockSpec((bm, DIM_OUT), lambda i: (i, 0)),
          pl.BlockSpec((bm, DIM_OUT), lambda i: (i, 0)),
          pl.BlockSpec((bm, 1), lambda i: (i, 0)),
          pl.BlockSpec((bm, 1), lambda i: (i, 0)),
          pl.BlockSpec((DIM_OUT, DIM_HID), lambda i: (0, 0)),
          pl.BlockSpec((1, DIM_HID), lambda i: (0, 0)),
          pl.BlockSpec((1, DIM_OUT), lambda i: (0, 0)),
      ],
      out_specs=pl.BlockSpec((bm, DIM_OUT), lambda i: (i, 0)),
      out_shape=jax.ShapeDtypeStruct((N_NODES, DIM_OUT), jnp.float32),
  )(op0, op1, c0, c1, Wout, b2r, boutr)


def kernel(x, edge_index, edge_weight, node_id, alpha, W1, b1, W2, b2,
           Wout, bout):
  wbits = lax.bitcast_convert_type(edge_weight, jnp.int32)
  npad_e = ETP - ET
  ep3 = jnp.stack([edge_index[0], edge_index[1], wbits]).reshape(
      3, NC * NS, ET)
  pads = jnp.stack([
      jnp.zeros((npad_e,), jnp.int32),
      N_NODES + (jnp.arange(npad_e, dtype=jnp.int32) % L),
      jnp.zeros((npad_e,), jnp.int32),
  ])
  pads3 = jnp.broadcast_to(pads[:, None, :], (3, NC * NS, npad_e))
  ep = jnp.concatenate([
      jnp.concatenate([ep3, pads3], axis=2).reshape(3, E_PACK),
      jnp.zeros((3, 2 * K), jnp.int32),
  ], axis=1)
  alpha_pad = jnp.zeros((ALPHA_PAD,), jnp.float32).at[: GENE_NUM + 2].set(
      alpha[:, 0])
  z128 = jnp.zeros((STRIPE, DIM_IN), jnp.float32)
  z32 = jnp.zeros((STRIPE, DIM_OUT), jnp.float32)
  z1 = jnp.zeros((STRIPE,), jnp.float32)

  p0, p1, cp0, cp1 = _sc_agg1(ep, node_id, alpha_pad, x, z128, z1)
  c0 = cp0.reshape(NPAD, 1)
  c1 = cp1.reshape(NPAD, 1)
  u = _tc_dense(p0, p1, c0, c1, W1, b1.reshape(1, -1), W2, Wout)
  op0, op1 = _sc_agg2(ep, node_id, alpha_pad, u, z32)
  return _tc_final(op0, op1, c0[:N_NODES], c1[:N_NODES], Wout,
                   b2.reshape(1, -1), bout.reshape(1, -1))


# speedup vs baseline: 1.1391x; 1.1391x over previous
import jax
import jax.numpy as jnp
from jax import lax
from jax.experimental import pallas as pl
from jax.experimental.pallas import tpu as pltpu
from jax.experimental.pallas import tpu_sc as plsc

N_NODES = 10000
GENE_NUM = 2000
N_EDGES = 320000
DIM_IN = 128
DIM_HID = 256
DIM_OUT = 32

NC = 2
NS = 16
L = 16

NPAD = 10240
STRIPE = NPAD // NS
K = 80
ET = N_EDGES // (NC * NS)
NB = ET // K
ALPHA_PAD = 2048


def _scale_rows(rows_ref, c_ref, n_rows, width):
  nch = width // L
  for t in range(n_rows // L):
    ch = c_ref[pl.ds(t * L, L)]
    for rr in range(L):
      cs = jnp.full((L,), ch[rr], dtype=jnp.float32)
      r = t * L + rr
      for j in range(nch):
        sl = pl.ds(j * L, L)
        rows_ref[r, sl] = rows_ref[r, sl] * cs


def _edge_coeffs(ebuf, nid_v, al_v, cv, dstb):
  for t in range(K // L):
    sl = pl.ds(t * L, L)
    sv = ebuf[0, sl]
    dv = ebuf[1, sl]
    w = plsc.bitcast(ebuf[2, sl], jnp.float32)
    sid = plsc.load_gather(nid_v, [sv])
    did = plsc.load_gather(nid_v, [dv])
    aidx = jnp.full((L,), GENE_NUM + 1, jnp.int32)
    aidx = jnp.where((sid >= 0) & (did < 0), sid, aidx)
    aidx = jnp.where((did >= 0) & (sid < 0), did, aidx)
    a = plsc.load_gather(al_v, [aidx])
    cv[sl] = w * a
    dstb[sl] = dv


def _pipeline(ep_hbm, feat_hbm, acc_sh, cnt_sh, nid_v, al_v, ebufs, dstbs,
              cv, ones_v, rows, sems_e, sems_g, sems_w, sems_c, width, base0):
  def ep_slice(b):
    return ep_hbm.at[:, pl.ds(base0 + b * K, K)]

  def fire_e(b, slot):
    pltpu.async_copy(ep_slice(b), ebufs[slot], sems_e[slot])

  def wait_e(slot):
    pltpu.make_async_copy(ep_slice(0), ebufs[slot], sems_e[slot]).wait()

  def fire_g(slot):
    pltpu.async_copy(feat_hbm.at[ebufs[slot].at[0]], rows[slot],
                     sems_g[slot])

  def wait_g(slot):
    pltpu.make_async_copy(feat_hbm.at[ebufs[slot].at[0]], rows[slot],
                          sems_g[slot]).wait()

  def fire_w(slot):
    pltpu.async_copy(rows[slot], acc_sh.at[dstbs[slot]], sems_w[slot],
                     add=True)

  def wait_w(slot):
    pltpu.make_async_copy(rows[slot], acc_sh.at[dstbs[slot]],
                          sems_w[slot]).wait()

  def fire_c(slot):
    pltpu.async_copy(ones_v, cnt_sh.at[dstbs[slot]], sems_c[slot], add=True)

  def wait_c(slot):
    pltpu.make_async_copy(ones_v, cnt_sh.at[dstbs[slot]],
                          sems_c[slot]).wait()

  def compute(slot):
    _edge_coeffs(ebufs[slot], nid_v, al_v, cv, dstbs[slot])
    if cnt_sh is not None:
      fire_c(slot)
    wait_g(slot)
    _scale_rows(rows[slot], cv, K, width)
    fire_w(slot)

  fire_e(0, 0)
  wait_e(0)
  fire_g(0)
  fire_e(1, 1)
  compute(0)
  fire_e(2, 0)
  wait_e(1)
  fire_g(1)
  compute(1)
  fire_e(3, 1)
  wait_e(0)
  wait_w(0)
  fire_g(0)

  def sub(b, slot):
    if cnt_sh is not None:
      wait_c(slot)
    compute(slot)
    fire_e(b + 2, slot)
    wait_e(1 - slot)
    wait_w(1 - slot)
    fire_g(1 - slot)

  def body(i, _):
    sub(2 * i + 2, 0)
    sub(2 * i + 3, 1)
    return 0

  lax.fori_loop(0, (NB - 3) // 2, body, 0)

  if cnt_sh is not None:
    wait_c(0)
  compute(0)
  wait_e(1)
  wait_w(0)
  wait_w(1)
  if cnt_sh is not None:
    wait_c(0)
    wait_c(1)


def _agg1_body(ep_hbm, nid_hbm, al_hbm, x_hbm, z128_hbm, z1_hbm,
               p0_hbm, p1_hbm, cp0_hbm, cp1_hbm,
               t1_sh, cnt_sh, ebuf0, ebuf1, dstb0, dstb1, cv, ones_v,
               rows0, rows1, nid_v, al_v,
               se0, se1, sg0, sg1, sw0, sw1, sc0, sc1):
  core = lax.axis_index("c")
  s = lax.axis_index("s")

  pltpu.sync_copy(z128_hbm, t1_sh.at[pl.ds(s * STRIPE, STRIPE)])
  pltpu.sync_copy(z1_hbm, cnt_sh.at[pl.ds(s * STRIPE, STRIPE)])
  for i in range(K // L):
    ones_v[pl.ds(i * L, L)] = jnp.ones((L,), jnp.float32)
  pltpu.sync_copy(nid_hbm, nid_v)
  pltpu.sync_copy(al_hbm, al_v)
  plsc.subcore_barrier()

  base0 = (core * NS + s) * ET
  _pipeline(ep_hbm, x_hbm, t1_sh, cnt_sh, nid_v, al_v, (ebuf0, ebuf1),
            (dstb0, dstb1), cv, ones_v, (rows0, rows1), (se0, se1),
            (sg0, sg1), (sw0, sw1), (sc0, sc1), DIM_IN, base0)
  plsc.subcore_barrier()

  stripe = pl.ds(s * STRIPE, STRIPE)

  @pl.when(core == 0)
  def _():
    pltpu.sync_copy(t1_sh.at[stripe], p0_hbm.at[stripe])
    pltpu.sync_copy(cnt_sh.at[stripe], cp0_hbm.at[stripe])

  @pl.when(core == 1)
  def _():
    pltpu.sync_copy(t1_sh.at[stripe], p1_hbm.at[stripe])
    pltpu.sync_copy(cnt_sh.at[stripe], cp1_hbm.at[stripe])


def _agg2_body(ep_hbm, nid_hbm, al_hbm, u_hbm, z32_hbm,
               op0_hbm, op1_hbm,
               out_sh, ebuf0, ebuf1, dstb0, dstb1, cv, ones_v, rows0, rows1,
               nid_v, al_v, se0, se1, sg0, sg1, sw0, sw1):
  core = lax.axis_index("c")
  s = lax.axis_index("s")

  pltpu.sync_copy(z32_hbm, out_sh.at[pl.ds(s * STRIPE, STRIPE)])
  pltpu.sync_copy(nid_hbm, nid_v)
  pltpu.sync_copy(al_hbm, al_v)
  plsc.subcore_barrier()

  base0 = (core * NS + s) * ET
  _pipeline(ep_hbm, u_hbm, out_sh, None, nid_v, al_v, (ebuf0, ebuf1),
            (dstb0, dstb1), cv, ones_v, (rows0, rows1), (se0, se1),
            (sg0, sg1), (sw0, sw1), None, DIM_OUT, base0)
  plsc.subcore_barrier()

  stripe = pl.ds(s * STRIPE, STRIPE)

  @pl.when(core == 0)
  def _():
    pltpu.sync_copy(out_sh.at[stripe], op0_hbm.at[stripe])

  @pl.when(core == 1)
  def _():
    pltpu.sync_copy(out_sh.at[stripe], op1_hbm.at[stripe])


def _sc_agg1(ep, node_id, alpha_pad, x, z128, z1):
  mesh = plsc.VectorSubcoreMesh(core_axis_name="c", subcore_axis_name="s")
  f = pl.kernel(
      _agg1_body,
      out_type=(
          jax.ShapeDtypeStruct((NPAD, DIM_IN), jnp.float32),
          jax.ShapeDtypeStruct((NPAD, DIM_IN), jnp.float32),
          jax.ShapeDtypeStruct((NPAD,), jnp.float32),
          jax.ShapeDtypeStruct((NPAD,), jnp.float32),
      ),
      mesh=mesh,
      scratch_types=[
          pltpu.VMEM_SHARED((NPAD, DIM_IN), jnp.float32),
          pltpu.VMEM_SHARED((NPAD,), jnp.float32),
          pltpu.VMEM((3, K), jnp.int32),
          pltpu.VMEM((3, K), jnp.int32),
          pltpu.VMEM((K,), jnp.int32),
          pltpu.VMEM((K,), jnp.int32),
          pltpu.VMEM((K,), jnp.float32),
          pltpu.VMEM((K,), jnp.float32),
          pltpu.VMEM((K, DIM_IN), jnp.float32),
          pltpu.VMEM((K, DIM_IN), jnp.float32),
          pltpu.VMEM((N_NODES,), jnp.int32),
          pltpu.VMEM((ALPHA_PAD,), jnp.float32),
      ] + [pltpu.SemaphoreType.DMA] * 8,
      compiler_params=pltpu.CompilerParams(needs_layout_passes=False,
                                           use_tc_tiling_on_sc=False),
  )
  return f(ep, node_id, alpha_pad, x, z128, z1)


def _sc_agg2(ep, node_id, alpha_pad, u, z32):
  mesh = plsc.VectorSubcoreMesh(core_axis_name="c", subcore_axis_name="s")
  f = pl.kernel(
      _agg2_body,
      out_type=(
          jax.ShapeDtypeStruct((NPAD, DIM_OUT), jnp.float32),
          jax.ShapeDtypeStruct((NPAD, DIM_OUT), jnp.float32),
      ),
      mesh=mesh,
      scratch_types=[
          pltpu.VMEM_SHARED((NPAD, DIM_OUT), jnp.float32),
          pltpu.VMEM((3, K), jnp.int32),
          pltpu.VMEM((3, K), jnp.int32),
          pltpu.VMEM((K,), jnp.int32),
          pltpu.VMEM((K,), jnp.int32),
          pltpu.VMEM((K,), jnp.float32),
          pltpu.VMEM((K,), jnp.float32),
          pltpu.VMEM((K, DIM_OUT), jnp.float32),
          pltpu.VMEM((K, DIM_OUT), jnp.float32),
          pltpu.VMEM((N_NODES,), jnp.int32),
          pltpu.VMEM((ALPHA_PAD,), jnp.float32),
      ] + [pltpu.SemaphoreType.DMA] * 6,
      compiler_params=pltpu.CompilerParams(needs_layout_passes=False,
                                           use_tc_tiling_on_sc=False),
  )
  return f(ep, node_id, alpha_pad, u, z32)


def _tc_dense_kern(p0_ref, p1_ref, c0_ref, c1_ref, w1_ref, b1_ref, w2_ref,
                   wo_ref, u_ref):
  r = 1.0 / jnp.maximum(c0_ref[...] + c1_ref[...], 1.0)
  t1 = (p0_ref[...] + p1_ref[...]) * r
  h2 = lax.dot_general(t1, w1_ref[...], (((1,), (1,)), ((), ())),
                       preferred_element_type=jnp.float32) + b1_ref[...]
  wc = lax.dot_general(wo_ref[...], w2_ref[...], (((1,), (0,)), ((), ())),
                       preferred_element_type=jnp.float32)
  u_ref[...] = lax.dot_general(h2, wc, (((1,), (1,)), ((), ())),
                               preferred_element_type=jnp.float32)


def _tc_dense(p0, p1, c0, c1, W1, b1r, W2, Wout):
  bm = 512
  grid = (NPAD // bm,)
  return pl.pallas_call(
      _tc_dense_kern,
      grid=grid,
      in_specs=[
          pl.BlockSpec((bm, DIM_IN), lambda i: (i, 0)),
          pl.BlockSpec((bm, DIM_IN), lambda i: (i, 0)),
          pl.BlockSpec((bm, 1), lambda i: (i, 0)),
          pl.BlockSpec((bm, 1), lambda i: (i, 0)),
          pl.BlockSpec((DIM_HID, DIM_IN), lambda i: (0, 0)),
          pl.BlockSpec((1, DIM_HID), lambda i: (0, 0)),
          pl.BlockSpec((DIM_HID, DIM_HID), lambda i: (0, 0)),
          pl.BlockSpec((DIM_OUT, DIM_HID), lambda i: (0, 0)),
      ],
      out_specs=pl.BlockSpec((bm, DIM_OUT), lambda i: (i, 0)),
      out_shape=jax.ShapeDtypeStruct((NPAD, DIM_OUT), jnp.float32),
  )(p0, p1, c0, c1, W1, b1r, W2, Wout)


def _tc_final_kern(p0_ref, p1_ref, c0_ref, c1_ref, wo_ref, b2_ref, bo_ref,
                   out_ref):
  r = 1.0 / jnp.maximum(c0_ref[...] + c1_ref[...], 1.0)
  bc = lax.dot_general(b2_ref[...], wo_ref[...], (((1,), (1,)), ((), ())),
                       preferred_element_type=jnp.float32)
  out_ref[...] = (p0_ref[...] + p1_ref[...]) * r + bc + bo_ref[...]


def _tc_final(op0, op1, c0, c1, Wout, b2r, boutr):
  bm = 400
  grid = (N_NODES // bm,)
  return pl.pallas_call(
      _tc_final_kern,
      grid=grid,
      in_specs=[
          pl.BlockSpec((bm, DIM_OUT), lambda i: (i, 0)),
          pl.BlockSpec((bm, DIM_OUT), lambda i: (i, 0)),
          pl.BlockSpec((bm, 1), lambda i: (i, 0)),
          pl.BlockSpec((bm, 1), lambda i: (i, 0)),
          pl.BlockSpec((DIM_OUT, DIM_HID), lambda i: (0, 0)),
          pl.BlockSpec((1, DIM_HID), lambda i: (0, 0)),
          pl.BlockSpec((1, DIM_OUT), lambda i: (0, 0)),
      ],
      out_specs=pl.BlockSpec((bm, DIM_OUT), lambda i: (i, 0)),
      out_shape=jax.ShapeDtypeStruct((N_NODES, DIM_OUT), jnp.float32),
  )(op0, op1, c0, c1, Wout, b2r, boutr)


def kernel(x, edge_index, edge_weight, node_id, alpha, W1, b1, W2, b2,
           Wout, bout):
  wbits = lax.bitcast_convert_type(edge_weight, jnp.int32)
  ep = jnp.zeros((3, N_EDGES + 2 * K), jnp.int32)
  ep = ep.at[:, :N_EDGES].set(
      jnp.stack([edge_index[0], edge_index[1], wbits]))
  alpha_pad = jnp.zeros((ALPHA_PAD,), jnp.float32).at[: GENE_NUM + 2].set(
      alpha[:, 0])
  z128 = jnp.zeros((STRIPE, DIM_IN), jnp.float32)
  z32 = jnp.zeros((STRIPE, DIM_OUT), jnp.float32)
  z1 = jnp.zeros((STRIPE,), jnp.float32)

  p0, p1, cp0, cp1 = _sc_agg1(ep, node_id, alpha_pad, x, z128, z1)
  c0 = cp0.reshape(NPAD, 1)
  c1 = cp1.reshape(NPAD, 1)
  u = _tc_dense(p0, p1, c0, c1, W1, b1.reshape(1, -1), W2, Wout)
  op0, op1 = _sc_agg2(ep, node_id, alpha_pad, u, z32)
  return _tc_final(op0, op1, c0[:N_NODES], c1[:N_NODES], Wout,
                   b2.reshape(1, -1), bout.reshape(1, -1))


# speedup vs baseline: 1.1539x; 1.0130x over previous
import jax
import jax.numpy as jnp
from jax import lax
from jax.experimental import pallas as pl
from jax.experimental.pallas import tpu as pltpu
from jax.experimental.pallas import tpu_sc as plsc

N_NODES = 10000
GENE_NUM = 2000
N_EDGES = 320000
DIM_IN = 128
DIM_HID = 256
DIM_OUT = 32

NC = 2
NS = 16
L = 16

NPAD = 10240
STRIPE = NPAD // NS
K = 80
ET = N_EDGES // (NC * NS)
NB = ET // K
ALPHA_PAD = 2048


def _scale_rows(rows_ref, c_ref, n_rows, width):
  nch = width // L
  for t in range(n_rows // L):
    ch = c_ref[pl.ds(t * L, L)]
    for rr in range(L):
      cs = jnp.full((L,), ch[rr], dtype=jnp.float32)
      r = t * L + rr
      for j in range(nch):
        sl = pl.ds(j * L, L)
        rows_ref[r, sl] = rows_ref[r, sl] * cs


def _scale_rows_bf(dst_ref, src_ref, c_ref, n_rows, width):
  for t in range(n_rows // L):
    ch = c_ref[pl.ds(t * L, L)]
    for rr in range(L):
      cs = jnp.full((L,), ch[rr], dtype=jnp.float32)
      r = t * L + rr
      for j in range(width // (2 * L)):
        pair = src_ref[r, pl.ds(j * 2 * L, 2 * L)]
        a, b = plsc.unpack(pair, format=plsc.PackFormat.INTERLEAVED,
                           preferred_element_type=jnp.float32)
        dst_ref[r, pl.ds(j * 2 * L, L)] = a * cs
        dst_ref[r, pl.ds(j * 2 * L + L, L)] = b * cs


def _edge_coeffs(ebuf, nid_v, al_v, cv, dstb):
  for t in range(K // L):
    sl = pl.ds(t * L, L)
    sv = ebuf[0, sl]
    dv = ebuf[1, sl]
    w = plsc.bitcast(ebuf[2, sl], jnp.float32)
    sid = plsc.load_gather(nid_v, [sv])
    did = plsc.load_gather(nid_v, [dv])
    aidx = jnp.full((L,), GENE_NUM + 1, jnp.int32)
    aidx = jnp.where((sid >= 0) & (did < 0), sid, aidx)
    aidx = jnp.where((did >= 0) & (sid < 0), did, aidx)
    a = plsc.load_gather(al_v, [aidx])
    cv[sl] = w * a
    dstb[sl] = dv


def _pipeline(ep_hbm, feat_hbm, acc_sh, cnt_sh, nid_v, al_v, ebufs, dstbs,
              cv, ones_v, rows, sems_e, sems_g, sems_w, sems_c, width, base0,
              rows_bf=None):
  def ep_slice(b):
    return ep_hbm.at[:, pl.ds(base0 + b * K, K)]

  def fire_e(b, slot):
    pltpu.async_copy(ep_slice(b), ebufs[slot], sems_e[slot])

  def wait_e(slot):
    pltpu.make_async_copy(ep_slice(0), ebufs[slot], sems_e[slot]).wait()

  gdst = rows if rows_bf is None else rows_bf

  def fire_g(slot):
    pltpu.async_copy(feat_hbm.at[ebufs[slot].at[0]], gdst[slot],
                     sems_g[slot])

  def wait_g(slot):
    pltpu.make_async_copy(feat_hbm.at[ebufs[slot].at[0]], gdst[slot],
                          sems_g[slot]).wait()

  def fire_w(slot):
    pltpu.async_copy(rows[slot], acc_sh.at[dstbs[slot]], sems_w[slot],
                     add=True)

  def wait_w(slot):
    pltpu.make_async_copy(rows[slot], acc_sh.at[dstbs[slot]],
                          sems_w[slot]).wait()

  def fire_c(slot):
    pltpu.async_copy(ones_v, cnt_sh.at[dstbs[slot]], sems_c[slot], add=True)

  def wait_c(slot):
    pltpu.make_async_copy(ones_v, cnt_sh.at[dstbs[slot]],
                          sems_c[slot]).wait()

  def compute(slot):
    _edge_coeffs(ebufs[slot], nid_v, al_v, cv, dstbs[slot])
    if cnt_sh is not None:
      fire_c(slot)
    wait_g(slot)
    if rows_bf is None:
      _scale_rows(rows[slot], cv, K, width)
    else:
      _scale_rows_bf(rows[slot], rows_bf[slot], cv, K, width)
    fire_w(slot)

  fire_e(0, 0)
  wait_e(0)
  fire_g(0)
  fire_e(1, 1)
  compute(0)
  fire_e(2, 0)
  wait_e(1)
  fire_g(1)
  compute(1)
  fire_e(3, 1)
  wait_e(0)
  wait_w(0)
  fire_g(0)

  def sub(b, slot):
    if cnt_sh is not None:
      wait_c(slot)
    compute(slot)
    fire_e(b + 2, slot)
    wait_e(1 - slot)
    wait_w(1 - slot)
    fire_g(1 - slot)

  def body(i, _):
    sub(2 * i + 2, 0)
    sub(2 * i + 3, 1)
    return 0

  lax.fori_loop(0, (NB - 3) // 2, body, 0)

  if cnt_sh is not None:
    wait_c(0)
  compute(0)
  wait_e(1)
  wait_w(0)
  wait_w(1)
  if cnt_sh is not None:
    wait_c(0)
    wait_c(1)


def _agg1_body(ep_hbm, nid_hbm, al_hbm, xbf_hbm, z128_hbm, z1_hbm,
               p0_hbm, p1_hbm, cp0_hbm, cp1_hbm,
               t1_sh, cnt_sh, ebuf0, ebuf1, dstb0, dstb1, cv, ones_v,
               rows0, rows1, rbf0, rbf1, nid_v, al_v,
               se0, se1, sg0, sg1, sw0, sw1, sc0, sc1):
  core = lax.axis_index("c")
  s = lax.axis_index("s")

  stripe = pl.ds(s * STRIPE, STRIPE)
  pltpu.sync_copy(z128_hbm, t1_sh.at[stripe])
  pltpu.sync_copy(z1_hbm, cnt_sh.at[stripe])
  for i in range(K // L):
    ones_v[pl.ds(i * L, L)] = jnp.ones((L,), jnp.float32)
  pltpu.sync_copy(nid_hbm, nid_v)
  pltpu.sync_copy(al_hbm, al_v)
  plsc.subcore_barrier()

  base0 = (core * NS + s) * ET
  _pipeline(ep_hbm, xbf_hbm, t1_sh, cnt_sh, nid_v, al_v, (ebuf0, ebuf1),
            (dstb0, dstb1), cv, ones_v, (rows0, rows1), (se0, se1),
            (sg0, sg1), (sw0, sw1), (sc0, sc1), DIM_IN, base0,
            rows_bf=(rbf0, rbf1))
  plsc.subcore_barrier()

  stripe = pl.ds(s * STRIPE, STRIPE)

  @pl.when(core == 0)
  def _():
    pltpu.sync_copy(t1_sh.at[stripe], p0_hbm.at[stripe])
    pltpu.sync_copy(cnt_sh.at[stripe], cp0_hbm.at[stripe])

  @pl.when(core == 1)
  def _():
    pltpu.sync_copy(t1_sh.at[stripe], p1_hbm.at[stripe])
    pltpu.sync_copy(cnt_sh.at[stripe], cp1_hbm.at[stripe])


def _agg2_body(ep_hbm, nid_hbm, al_hbm, u_hbm, z32_hbm,
               op0_hbm, op1_hbm,
               out_sh, ebuf0, ebuf1, dstb0, dstb1, cv, ones_v, rows0, rows1,
               nid_v, al_v, se0, se1, sg0, sg1, sw0, sw1):
  core = lax.axis_index("c")
  s = lax.axis_index("s")

  pltpu.sync_copy(z32_hbm, out_sh.at[pl.ds(s * STRIPE, STRIPE)])
  pltpu.sync_copy(nid_hbm, nid_v)
  pltpu.sync_copy(al_hbm, al_v)
  plsc.subcore_barrier()

  base0 = (core * NS + s) * ET
  _pipeline(ep_hbm, u_hbm, out_sh, None, nid_v, al_v, (ebuf0, ebuf1),
            (dstb0, dstb1), cv, ones_v, (rows0, rows1), (se0, se1),
            (sg0, sg1), (sw0, sw1), None, DIM_OUT, base0)
  plsc.subcore_barrier()

  stripe = pl.ds(s * STRIPE, STRIPE)

  @pl.when(core == 0)
  def _():
    pltpu.sync_copy(out_sh.at[stripe], op0_hbm.at[stripe])

  @pl.when(core == 1)
  def _():
    pltpu.sync_copy(out_sh.at[stripe], op1_hbm.at[stripe])


def _sc_agg1(ep, node_id, alpha_pad, xbf, z128, z1):
  mesh = plsc.VectorSubcoreMesh(core_axis_name="c", subcore_axis_name="s")
  f = pl.kernel(
      _agg1_body,
      out_type=(
          jax.ShapeDtypeStruct((NPAD, DIM_IN), jnp.float32),
          jax.ShapeDtypeStruct((NPAD, DIM_IN), jnp.float32),
          jax.ShapeDtypeStruct((NPAD,), jnp.float32),
          jax.ShapeDtypeStruct((NPAD,), jnp.float32),
      ),
      mesh=mesh,
      scratch_types=[
          pltpu.VMEM_SHARED((NPAD, DIM_IN), jnp.float32),
          pltpu.VMEM_SHARED((NPAD,), jnp.float32),
          pltpu.VMEM((3, K), jnp.int32),
          pltpu.VMEM((3, K), jnp.int32),
          pltpu.VMEM((K,), jnp.int32),
          pltpu.VMEM((K,), jnp.int32),
          pltpu.VMEM((K,), jnp.float32),
          pltpu.VMEM((K,), jnp.float32),
          pltpu.VMEM((K, DIM_IN), jnp.float32),
          pltpu.VMEM((K, DIM_IN), jnp.float32),
          pltpu.VMEM((K, DIM_IN), jnp.bfloat16),
          pltpu.VMEM((K, DIM_IN), jnp.bfloat16),
          pltpu.VMEM((N_NODES,), jnp.int32),
          pltpu.VMEM((ALPHA_PAD,), jnp.float32),
      ] + [pltpu.SemaphoreType.DMA] * 8,
      compiler_params=pltpu.CompilerParams(needs_layout_passes=False,
                                           use_tc_tiling_on_sc=False),
  )
  return f(ep, node_id, alpha_pad, xbf, z128, z1)


def _sc_agg2(ep, node_id, alpha_pad, u, z32):
  mesh = plsc.VectorSubcoreMesh(core_axis_name="c", subcore_axis_name="s")
  f = pl.kernel(
      _agg2_body,
      out_type=(
          jax.ShapeDtypeStruct((NPAD, DIM_OUT), jnp.float32),
          jax.ShapeDtypeStruct((NPAD, DIM_OUT), jnp.float32),
      ),
      mesh=mesh,
      scratch_types=[
          pltpu.VMEM_SHARED((NPAD, DIM_OUT), jnp.float32),
          pltpu.VMEM((3, K), jnp.int32),
          pltpu.VMEM((3, K), jnp.int32),
          pltpu.VMEM((K,), jnp.int32),
          pltpu.VMEM((K,), jnp.int32),
          pltpu.VMEM((K,), jnp.float32),
          pltpu.VMEM((K,), jnp.float32),
          pltpu.VMEM((K, DIM_OUT), jnp.float32),
          pltpu.VMEM((K, DIM_OUT), jnp.float32),
          pltpu.VMEM((N_NODES,), jnp.int32),
          pltpu.VMEM((ALPHA_PAD,), jnp.float32),
      ] + [pltpu.SemaphoreType.DMA] * 6,
      compiler_params=pltpu.CompilerParams(needs_layout_passes=False,
                                           use_tc_tiling_on_sc=False),
  )
  return f(ep, node_id, alpha_pad, u, z32)


def _tc_dense_kern(p0_ref, p1_ref, c0_ref, c1_ref, w1_ref, b1_ref, w2_ref,
                   wo_ref, u_ref):
  r = 1.0 / jnp.maximum(c0_ref[...] + c1_ref[...], 1.0)
  t1 = (p0_ref[...] + p1_ref[...]) * r
  h2 = lax.dot_general(t1, w1_ref[...], (((1,), (1,)), ((), ())),
                       preferred_element_type=jnp.float32) + b1_ref[...]
  wc = lax.dot_general(wo_ref[...], w2_ref[...], (((1,), (0,)), ((), ())),
                       preferred_element_type=jnp.float32)
  u_ref[...] = lax.dot_general(h2, wc, (((1,), (1,)), ((), ())),
                               preferred_element_type=jnp.float32)


def _tc_dense(p0, p1, c0, c1, W1, b1r, W2, Wout):
  bm = 512
  grid = (NPAD // bm,)
  return pl.pallas_call(
      _tc_dense_kern,
      grid=grid,
      in_specs=[
          pl.BlockSpec((bm, DIM_IN), lambda i: (i, 0)),
          pl.BlockSpec((bm, DIM_IN), lambda i: (i, 0)),
          pl.BlockSpec((bm, 1), lambda i: (i, 0)),
          pl.BlockSpec((bm, 1), lambda i: (i, 0)),
          pl.BlockSpec((DIM_HID, DIM_IN), lambda i: (0, 0)),
          pl.BlockSpec((1, DIM_HID), lambda i: (0, 0)),
          pl.BlockSpec((DIM_HID, DIM_HID), lambda i: (0, 0)),
          pl.BlockSpec((DIM_OUT, DIM_HID), lambda i: (0, 0)),
      ],
      out_specs=pl.BlockSpec((bm, DIM_OUT), lambda i: (i, 0)),
      out_shape=jax.ShapeDtypeStruct((NPAD, DIM_OUT), jnp.float32),
  )(p0, p1, c0, c1, W1, b1r, W2, Wout)


def _tc_final_kern(p0_ref, p1_ref, c0_ref, c1_ref, wo_ref, b2_ref, bo_ref,
                   out_ref):
  r = 1.0 / jnp.maximum(c0_ref[...] + c1_ref[...], 1.0)
  bc = lax.dot_general(b2_ref[...], wo_ref[...], (((1,), (1,)), ((), ())),
                       preferred_element_type=jnp.float32)
  out_ref[...] = (p0_ref[...] + p1_ref[...]) * r + bc + bo_ref[...]


def _tc_final(op0, op1, c0, c1, Wout, b2r, boutr):
  bm = 400
  grid = (N_NODES // bm,)
  return pl.pallas_call(
      _tc_final_kern,
      grid=grid,
      in_specs=[
          pl.BlockSpec((bm, DIM_OUT), lambda i: (i, 0)),
          pl.BlockSpec((bm, DIM_OUT), lambda i: (i, 0)),
          pl.BlockSpec((bm, 1), lambda i: (i, 0)),
          pl.BlockSpec((bm, 1), lambda i: (i, 0)),
          pl.BlockSpec((DIM_OUT, DIM_HID), lambda i: (0, 0)),
          pl.BlockSpec((1, DIM_HID), lambda i: (0, 0)),
          pl.BlockSpec((1, DIM_OUT), lambda i: (0, 0)),
      ],
      out_specs=pl.BlockSpec((bm, DIM_OUT), lambda i: (i, 0)),
      out_shape=jax.ShapeDtypeStruct((N_NODES, DIM_OUT), jnp.float32),
  )(op0, op1, c0, c1, Wout, b2r, boutr)


def kernel(x, edge_index, edge_weight, node_id, alpha, W1, b1, W2, b2,
           Wout, bout):
  wbits = lax.bitcast_convert_type(edge_weight, jnp.int32)
  ep = jnp.zeros((3, N_EDGES + 2 * K), jnp.int32)
  ep = ep.at[:, :N_EDGES].set(
      jnp.stack([edge_index[0], edge_index[1], wbits]))
  alpha_pad = jnp.zeros((ALPHA_PAD,), jnp.float32).at[: GENE_NUM + 2].set(
      alpha[:, 0])
  z128 = jnp.zeros((STRIPE, DIM_IN), jnp.float32)
  z32 = jnp.zeros((STRIPE, DIM_OUT), jnp.float32)
  z1 = jnp.zeros((STRIPE,), jnp.float32)

  i16 = jnp.arange(L, dtype=jnp.int32)
  blk = jnp.stack([i16, L + i16], axis=1).reshape(2 * L)
  perm = jnp.concatenate([g * 2 * L + blk for g in range(DIM_IN // (2 * L))])
  xbf = jnp.zeros((NPAD, DIM_IN), jnp.bfloat16).at[:N_NODES].set(
      x.astype(jnp.bfloat16)[:, perm])

  p0, p1, cp0, cp1 = _sc_agg1(ep, node_id, alpha_pad, xbf, z128, z1)
  c0 = cp0.reshape(NPAD, 1)
  c1 = cp1.reshape(NPAD, 1)
  u = _tc_dense(p0, p1, c0, c1, W1, b1.reshape(1, -1), W2, Wout)
  op0, op1 = _sc_agg2(ep, node_id, alpha_pad, u, z32)
  return _tc_final(op0, op1, c0[:N_NODES], c1[:N_NODES], Wout,
                   b2.reshape(1, -1), bout.reshape(1, -1))


# speedup vs baseline: 1.1563x; 1.0021x over previous
import jax
import jax.numpy as jnp
from jax import lax
from jax.experimental import pallas as pl
from jax.experimental.pallas import tpu as pltpu
from jax.experimental.pallas import tpu_sc as plsc

N_NODES = 10000
GENE_NUM = 2000
N_EDGES = 320000
DIM_IN = 128
DIM_HID = 256
DIM_OUT = 32

NC = 2
NS = 16
L = 16

NPAD = 10240
STRIPE = NPAD // NS
K = 80
ET = N_EDGES // (NC * NS)
NB = ET // K
ALPHA_PAD = 2048


def _scale_rows(rows_ref, c_ref, n_rows, width):
  nch = width // L
  for t in range(n_rows // L):
    ch = c_ref[pl.ds(t * L, L)]
    for rr in range(L):
      cs = jnp.full((L,), ch[rr], dtype=jnp.float32)
      r = t * L + rr
      for j in range(nch):
        sl = pl.ds(j * L, L)
        rows_ref[r, sl] = rows_ref[r, sl] * cs


def _scale_rows_bf(dst_ref, src_ref, c_ref, n_rows, width):
  for t in range(n_rows // L):
    ch = c_ref[pl.ds(t * L, L)]
    for rr in range(L):
      cs = jnp.full((L,), ch[rr], dtype=jnp.float32)
      r = t * L + rr
      for j in range(width // (2 * L)):
        pair = src_ref[r, pl.ds(j * 2 * L, 2 * L)]
        a, b = plsc.unpack(pair, format=plsc.PackFormat.INTERLEAVED,
                           preferred_element_type=jnp.float32)
        dst_ref[r, pl.ds(j * 2 * L, L)] = a * cs
        dst_ref[r, pl.ds(j * 2 * L + L, L)] = b * cs


def _edge_coeffs(ebuf, nid_v, al_v, cv, dstb):
  for t in range(K // L):
    sl = pl.ds(t * L, L)
    sv = ebuf[0, sl]
    dv = ebuf[1, sl]
    w = plsc.bitcast(ebuf[2, sl], jnp.float32)
    sid = plsc.load_gather(nid_v, [sv])
    did = plsc.load_gather(nid_v, [dv])
    aidx = jnp.full((L,), GENE_NUM + 1, jnp.int32)
    aidx = jnp.where((sid >= 0) & (did < 0), sid, aidx)
    aidx = jnp.where((did >= 0) & (sid < 0), did, aidx)
    a = plsc.load_gather(al_v, [aidx])
    cv[sl] = w * a
    dstb[sl] = dv


def _pipeline(ep_hbm, feat_hbm, acc_sh, cnt_sh, nid_v, al_v, ebufs, dstbs,
              cv, ones_v, rows, sems_e, sems_g, sems_w, sems_c, width, base0,
              rows_bf=None):
  def ep_slice(b):
    return ep_hbm.at[:, pl.ds(base0 + b * K, K)]

  def fire_e(b, slot):
    pltpu.async_copy(ep_slice(b), ebufs[slot], sems_e[slot])

  def wait_e(slot):
    pltpu.make_async_copy(ep_slice(0), ebufs[slot], sems_e[slot]).wait()

  gdst = rows if rows_bf is None else rows_bf
  H = K // 2

  def fire_g(slot):
    pltpu.async_copy(feat_hbm.at[ebufs[slot].at[0, pl.ds(0, H)]],
                     gdst[slot].at[pl.ds(0, H)], sems_g[slot])
    pltpu.async_copy(feat_hbm.at[ebufs[slot].at[0, pl.ds(H, H)]],
                     gdst[slot].at[pl.ds(H, H)], sems_g[slot])

  def wait_g(slot):
    pltpu.make_async_copy(feat_hbm.at[ebufs[slot].at[0, pl.ds(0, H)]],
                          gdst[slot].at[pl.ds(0, H)], sems_g[slot]).wait()
    pltpu.make_async_copy(feat_hbm.at[ebufs[slot].at[0, pl.ds(H, H)]],
                          gdst[slot].at[pl.ds(H, H)], sems_g[slot]).wait()

  def fire_w(slot):
    pltpu.async_copy(rows[slot], acc_sh.at[dstbs[slot]], sems_w[slot],
                     add=True)

  def wait_w(slot):
    pltpu.make_async_copy(rows[slot], acc_sh.at[dstbs[slot]],
                          sems_w[slot]).wait()

  def fire_c(slot):
    pltpu.async_copy(ones_v, cnt_sh.at[dstbs[slot]], sems_c[slot], add=True)

  def wait_c(slot):
    pltpu.make_async_copy(ones_v, cnt_sh.at[dstbs[slot]],
                          sems_c[slot]).wait()

  def compute(slot):
    _edge_coeffs(ebufs[slot], nid_v, al_v, cv, dstbs[slot])
    if cnt_sh is not None:
      fire_c(slot)
    wait_g(slot)
    if rows_bf is None:
      _scale_rows(rows[slot], cv, K, width)
    else:
      _scale_rows_bf(rows[slot], rows_bf[slot], cv, K, width)
    fire_w(slot)

  fire_e(0, 0)
  wait_e(0)
  fire_g(0)
  fire_e(1, 1)
  compute(0)
  fire_e(2, 0)
  wait_e(1)
  fire_g(1)
  compute(1)
  fire_e(3, 1)
  wait_e(0)
  wait_w(0)
  fire_g(0)

  def sub(b, slot):
    if cnt_sh is not None:
      wait_c(slot)
    compute(slot)
    fire_e(b + 2, slot)
    wait_e(1 - slot)
    wait_w(1 - slot)
    fire_g(1 - slot)

  def body(i, _):
    sub(2 * i + 2, 0)
    sub(2 * i + 3, 1)
    return 0

  lax.fori_loop(0, (NB - 3) // 2, body, 0)

  if cnt_sh is not None:
    wait_c(0)
  compute(0)
  wait_e(1)
  wait_w(0)
  wait_w(1)
  if cnt_sh is not None:
    wait_c(0)
    wait_c(1)


def _agg1_body(ep_hbm, nid_hbm, al_hbm, xbf_hbm, z128_hbm, z1_hbm,
               p0_hbm, p1_hbm, cp0_hbm, cp1_hbm,
               t1_sh, cnt_sh, ebuf0, ebuf1, dstb0, dstb1, cv, ones_v,
               rows0, rows1, rbf0, rbf1, nid_v, al_v,
               se0, se1, sg0, sg1, sw0, sw1, sc0, sc1):
  core = lax.axis_index("c")
  s = lax.axis_index("s")

  stripe = pl.ds(s * STRIPE, STRIPE)
  pltpu.sync_copy(z128_hbm, t1_sh.at[stripe])
  pltpu.sync_copy(z1_hbm, cnt_sh.at[stripe])
  for i in range(K // L):
    ones_v[pl.ds(i * L, L)] = jnp.ones((L,), jnp.float32)
  pltpu.sync_copy(nid_hbm, nid_v)
  pltpu.sync_copy(al_hbm, al_v)
  plsc.subcore_barrier()

  base0 = (core * NS + s) * ET
  _pipeline(ep_hbm, xbf_hbm, t1_sh, cnt_sh, nid_v, al_v, (ebuf0, ebuf1),
            (dstb0, dstb1), cv, ones_v, (rows0, rows1), (se0, se1),
            (sg0, sg1), (sw0, sw1), (sc0, sc1), DIM_IN, base0,
            rows_bf=(rbf0, rbf1))
  plsc.subcore_barrier()

  stripe = pl.ds(s * STRIPE, STRIPE)

  @pl.when(core == 0)
  def _():
    pltpu.sync_copy(t1_sh.at[stripe], p0_hbm.at[stripe])
    pltpu.sync_copy(cnt_sh.at[stripe], cp0_hbm.at[stripe])

  @pl.when(core == 1)
  def _():
    pltpu.sync_copy(t1_sh.at[stripe], p1_hbm.at[stripe])
    pltpu.sync_copy(cnt_sh.at[stripe], cp1_hbm.at[stripe])


def _agg2_body(ep_hbm, nid_hbm, al_hbm, u_hbm, z32_hbm,
               op0_hbm, op1_hbm,
               out_sh, ebuf0, ebuf1, dstb0, dstb1, cv, ones_v, rows0, rows1,
               nid_v, al_v, se0, se1, sg0, sg1, sw0, sw1):
  core = lax.axis_index("c")
  s = lax.axis_index("s")

  pltpu.sync_copy(z32_hbm, out_sh.at[pl.ds(s * STRIPE, STRIPE)])
  pltpu.sync_copy(nid_hbm, nid_v)
  pltpu.sync_copy(al_hbm, al_v)
  plsc.subcore_barrier()

  base0 = (core * NS + s) * ET
  _pipeline(ep_hbm, u_hbm, out_sh, None, nid_v, al_v, (ebuf0, ebuf1),
            (dstb0, dstb1), cv, ones_v, (rows0, rows1), (se0, se1),
            (sg0, sg1), (sw0, sw1), None, DIM_OUT, base0)
  plsc.subcore_barrier()

  stripe = pl.ds(s * STRIPE, STRIPE)

  @pl.when(core == 0)
  def _():
    pltpu.sync_copy(out_sh.at[stripe], op0_hbm.at[stripe])

  @pl.when(core == 1)
  def _():
    pltpu.sync_copy(out_sh.at[stripe], op1_hbm.at[stripe])


def _sc_agg1(ep, node_id, alpha_pad, xbf, z128, z1):
  mesh = plsc.VectorSubcoreMesh(core_axis_name="c", subcore_axis_name="s")
  f = pl.kernel(
      _agg1_body,
      out_type=(
          jax.ShapeDtypeStruct((NPAD, DIM_IN), jnp.float32),
          jax.ShapeDtypeStruct((NPAD, DIM_IN), jnp.float32),
          jax.ShapeDtypeStruct((NPAD,), jnp.float32),
          jax.ShapeDtypeStruct((NPAD,), jnp.float32),
      ),
      mesh=mesh,
      scratch_types=[
          pltpu.VMEM_SHARED((NPAD, DIM_IN), jnp.float32),
          pltpu.VMEM_SHARED((NPAD,), jnp.float32),
          pltpu.VMEM((3, K), jnp.int32),
          pltpu.VMEM((3, K), jnp.int32),
          pltpu.VMEM((K,), jnp.int32),
          pltpu.VMEM((K,), jnp.int32),
          pltpu.VMEM((K,), jnp.float32),
          pltpu.VMEM((K,), jnp.float32),
          pltpu.VMEM((K, DIM_IN), jnp.float32),
          pltpu.VMEM((K, DIM_IN), jnp.float32),
          pltpu.VMEM((K, DIM_IN), jnp.bfloat16),
          pltpu.VMEM((K, DIM_IN), jnp.bfloat16),
          pltpu.VMEM((N_NODES,), jnp.int32),
          pltpu.VMEM((ALPHA_PAD,), jnp.float32),
      ] + [pltpu.SemaphoreType.DMA] * 8,
      compiler_params=pltpu.CompilerParams(needs_layout_passes=False,
                                           use_tc_tiling_on_sc=False),
  )
  return f(ep, node_id, alpha_pad, xbf, z128, z1)


def _sc_agg2(ep, node_id, alpha_pad, u, z32):
  mesh = plsc.VectorSubcoreMesh(core_axis_name="c", subcore_axis_name="s")
  f = pl.kernel(
      _agg2_body,
      out_type=(
          jax.ShapeDtypeStruct((NPAD, DIM_OUT), jnp.float32),
          jax.ShapeDtypeStruct((NPAD, DIM_OUT), jnp.float32),
      ),
      mesh=mesh,
      scratch_types=[
          pltpu.VMEM_SHARED((NPAD, DIM_OUT), jnp.float32),
          pltpu.VMEM((3, K), jnp.int32),
          pltpu.VMEM((3, K), jnp.int32),
          pltpu.VMEM((K,), jnp.int32),
          pltpu.VMEM((K,), jnp.int32),
          pltpu.VMEM((K,), jnp.float32),
          pltpu.VMEM((K,), jnp.float32),
          pltpu.VMEM((K, DIM_OUT), jnp.float32),
          pltpu.VMEM((K, DIM_OUT), jnp.float32),
          pltpu.VMEM((N_NODES,), jnp.int32),
          pltpu.VMEM((ALPHA_PAD,), jnp.float32),
      ] + [pltpu.SemaphoreType.DMA] * 6,
      compiler_params=pltpu.CompilerParams(needs_layout_passes=False,
                                           use_tc_tiling_on_sc=False),
  )
  return f(ep, node_id, alpha_pad, u, z32)


def _tc_dense_kern(p0_ref, p1_ref, c0_ref, c1_ref, w1_ref, b1_ref, w2_ref,
                   wo_ref, u_ref):
  r = 1.0 / jnp.maximum(c0_ref[...] + c1_ref[...], 1.0)
  t1 = (p0_ref[...] + p1_ref[...]) * r
  h2 = lax.dot_general(t1, w1_ref[...], (((1,), (1,)), ((), ())),
                       preferred_element_type=jnp.float32) + b1_ref[...]
  wc = lax.dot_general(wo_ref[...], w2_ref[...], (((1,), (0,)), ((), ())),
                       preferred_element_type=jnp.float32)
  u_ref[...] = lax.dot_general(h2, wc, (((1,), (1,)), ((), ())),
                               preferred_element_type=jnp.float32)


def _tc_dense(p0, p1, c0, c1, W1, b1r, W2, Wout):
  bm = 512
  grid = (NPAD // bm,)
  return pl.pallas_call(
      _tc_dense_kern,
      grid=grid,
      in_specs=[
          pl.BlockSpec((bm, DIM_IN), lambda i: (i, 0)),
          pl.BlockSpec((bm, DIM_IN), lambda i: (i, 0)),
          pl.BlockSpec((bm, 1), lambda i: (i, 0)),
          pl.BlockSpec((bm, 1), lambda i: (i, 0)),
          pl.BlockSpec((DIM_HID, DIM_IN), lambda i: (0, 0)),
          pl.BlockSpec((1, DIM_HID), lambda i: (0, 0)),
          pl.BlockSpec((DIM_HID, DIM_HID), lambda i: (0, 0)),
          pl.BlockSpec((DIM_OUT, DIM_HID), lambda i: (0, 0)),
      ],
      out_specs=pl.BlockSpec((bm, DIM_OUT), lambda i: (i, 0)),
      out_shape=jax.ShapeDtypeStruct((NPAD, DIM_OUT), jnp.float32),
  )(p0, p1, c0, c1, W1, b1r, W2, Wout)


def _tc_final_kern(p0_ref, p1_ref, c0_ref, c1_ref, wo_ref, b2_ref, bo_ref,
                   out_ref):
  r = 1.0 / jnp.maximum(c0_ref[...] + c1_ref[...], 1.0)
  bc = lax.dot_general(b2_ref[...], wo_ref[...], (((1,), (1,)), ((), ())),
                       preferred_element_type=jnp.float32)
  out_ref[...] = (p0_ref[...] + p1_ref[...]) * r + bc + bo_ref[...]


def _tc_final(op0, op1, c0, c1, Wout, b2r, boutr):
  bm = 400
  grid = (N_NODES // bm,)
  return pl.pallas_call(
      _tc_final_kern,
      grid=grid,
      in_specs=[
          pl.BlockSpec((bm, DIM_OUT), lambda i: (i, 0)),
          pl.BlockSpec((bm, DIM_OUT), lambda i: (i, 0)),
          pl.BlockSpec((bm, 1), lambda i: (i, 0)),
          pl.BlockSpec((bm, 1), lambda i: (i, 0)),
          pl.BlockSpec((DIM_OUT, DIM_HID), lambda i: (0, 0)),
          pl.BlockSpec((1, DIM_HID), lambda i: (0, 0)),
          pl.BlockSpec((1, DIM_OUT), lambda i: (0, 0)),
      ],
      out_specs=pl.BlockSpec((bm, DIM_OUT), lambda i: (i, 0)),
      out_shape=jax.ShapeDtypeStruct((N_NODES, DIM_OUT), jnp.float32),
  )(op0, op1, c0, c1, Wout, b2r, boutr)


def kernel(x, edge_index, edge_weight, node_id, alpha, W1, b1, W2, b2,
           Wout, bout):
  wbits = lax.bitcast_convert_type(edge_weight, jnp.int32)
  ep = jnp.zeros((3, N_EDGES + 2 * K), jnp.int32)
  ep = ep.at[:, :N_EDGES].set(
      jnp.stack([edge_index[0], edge_index[1], wbits]))
  alpha_pad = jnp.zeros((ALPHA_PAD,), jnp.float32).at[: GENE_NUM + 2].set(
      alpha[:, 0])
  z128 = jnp.zeros((STRIPE, DIM_IN), jnp.float32)
  z32 = jnp.zeros((STRIPE, DIM_OUT), jnp.float32)
  z1 = jnp.zeros((STRIPE,), jnp.float32)

  i16 = jnp.arange(L, dtype=jnp.int32)
  blk = jnp.stack([i16, L + i16], axis=1).reshape(2 * L)
  perm = jnp.concatenate([g * 2 * L + blk for g in range(DIM_IN // (2 * L))])
  xbf = jnp.zeros((NPAD, DIM_IN), jnp.bfloat16).at[:N_NODES].set(
      x.astype(jnp.bfloat16)[:, perm])

  p0, p1, cp0, cp1 = _sc_agg1(ep, node_id, alpha_pad, xbf, z128, z1)
  c0 = cp0.reshape(NPAD, 1)
  c1 = cp1.reshape(NPAD, 1)
  u = _tc_dense(p0, p1, c0, c1, W1, b1.reshape(1, -1), W2, Wout)
  op0, op1 = _sc_agg2(ep, node_id, alpha_pad, u, z32)
  return _tc_final(op0, op1, c0[:N_NODES], c1[:N_NODES], Wout,
                   b2.reshape(1, -1), bout.reshape(1, -1))
